# Initial kernel scaffold; baseline (speedup 1.0000x reference)
import functools

import jax
import jax.numpy as jnp
from jax import lax
from jax.experimental import pallas as pl
from jax.experimental.pallas import tpu as pltpu
from jax.experimental.pallas import tpu_sc as plsc

NC, NS, L = 2, 16, 16
NW = NC * NS

B = 4
HF = 512
OC = 16
H0 = 128
NUM_CLASS = 21
NUM_OLD = 16

_mesh = lambda: plsc.VectorSubcoreMesh(core_axis_name="c", subcore_axis_name="s")


def _dsum_body(f_ref, g_ref, o_ref):
    c = pl.program_id(1)

    @pl.when(c == 0)
    def _():
        o_ref[...] = jnp.zeros_like(o_ref)

    x = f_ref[...] - g_ref[...]
    o_ref[...] += jnp.sum(x * x, axis=1, keepdims=True)


def _dsum(f, f_old, c_blk):
    b, c, h, w = f.shape
    hw = h * w
    f2 = f.reshape(b, c, hw)
    g2 = f_old.reshape(b, c, hw)
    out = pl.pallas_call(
        _dsum_body,
        grid=(b, c // c_blk),
        in_specs=[
            pl.BlockSpec((1, c_blk, hw), lambda i, j: (i, j, 0)),
            pl.BlockSpec((1, c_blk, hw), lambda i, j: (i, j, 0)),
        ],
        out_specs=pl.BlockSpec((1, 1, hw), lambda i, j: (i, 0, 0)),
        out_shape=jax.ShapeDtypeStruct((b, 1, hw), jnp.float32),
        compiler_params=pltpu.CompilerParams(
            dimension_semantics=("parallel", "arbitrary")),
    )(f2, g2)
    return out.reshape(b, hw)


@functools.partial(
    pl.kernel,
    out_type=jax.ShapeDtypeStruct((B * H0, H0), jnp.int32),
    mesh=_mesh(),
    scratch_types=[
        pltpu.VMEM((OC, HF), jnp.float32),
        pltpu.VMEM((1, HF), jnp.int32),
        pltpu.VMEM((1, H0), jnp.int32),
        pltpu.SemaphoreType.DMA,
        pltpu.SemaphoreType.DMA,
    ],
)
def _pseudo_kernel(oo_hbm, lab_hbm, out_hbm, rows_v, lab_v, ps_v, sem1, sem2):
    wid = lax.axis_index("s") * NC + lax.axis_index("c")
    iota = lax.iota(jnp.int32, (L,))
    ntask = (B * H0) // NW

    def task(k, carry):
        t = wid * ntask + k
        b = t // H0
        i = t % H0
        ridx = b * (OC * HF) + 4 * i + HF * iota
        cp1 = pltpu.async_copy(oo_hbm.at[ridx], rows_v, sem1)
        cp2 = pltpu.async_copy(lab_hbm.at[pl.ds(b * HF + 4 * i, 1)], lab_v, sem2)
        cp1.wait()
        cp2.wait()
        for j0 in range(0, H0, L):
            cols = 4 * (j0 + iota)
            best = None
            bidx = jnp.zeros((L,), jnp.int32)
            for c in range(OC):
                rv = plsc.load_gather(
                    rows_v, [jnp.full((L,), c, jnp.int32), cols])
                v = jnp.where(rv < 0.5, jnp.float32(0.0), rv)
                if c == 0:
                    best = v
                else:
                    take = v > best
                    best = jnp.where(take, v, best)
                    bidx = jnp.where(take, c, bidx)
            lv = plsc.load_gather(lab_v, [jnp.zeros((L,), jnp.int32), cols])
            ps_v[0, pl.ds(j0, L)] = jnp.where(lv == 0, bidx, lv)
        pltpu.sync_copy(ps_v, out_hbm.at[pl.ds(t, 1)])
        return carry

    lax.fori_loop(0, ntask, task, 0)


@functools.partial(
    pl.kernel,
    out_type=(
        jax.ShapeDtypeStruct((NW * 128,), jnp.float32),
        jax.ShapeDtypeStruct((NW * 128,), jnp.float32),
    ),
    mesh=_mesh(),
    scratch_types=[
        pltpu.VMEM((H0 * H0,), jnp.int32),
        pltpu.VMEM((2048,), jnp.float32),
        pltpu.VMEM((128,), jnp.float32),
        pltpu.VMEM((128,), jnp.float32),
    ],
)
def _bin_kernel(ps_hbm, d0_hbm, d1_hbm, d2_hbm, d3_hbm,
                s_hbm, n_hbm, ps_v, d_v, s_v, n_v):
    wid = lax.axis_index("s") * NC + lax.axis_index("c")
    b = wid // 8
    seg = wid % 8
    zero = jnp.zeros((L,), jnp.float32)
    ones = jnp.ones((L,), jnp.float32)
    iota = lax.iota(jnp.int32, (L,))
    for j in range(8):
        s_v[pl.ds(j * L, L)] = zero
        n_v[pl.ds(j * L, L)] = zero
    pltpu.sync_copy(ps_hbm.at[b], ps_v)

    pltpu.sync_copy(d0_hbm.at[b, pl.ds(seg * 2048, 2048)], d_v)

    def s0_body(v, carry):
        p = ps_v[pl.ds(seg * 2048 + v * L, L)]
        d = d_v[pl.ds(v * L, L)]
        plsc.addupdate_scatter(s_v, [p], d)
        plsc.addupdate_scatter(n_v, [p], ones)
        return carry

    lax.fori_loop(0, 2048 // L, s0_body, 0)

    pltpu.sync_copy(d1_hbm.at[b, pl.ds(seg * 512, 512)], d_v.at[pl.ds(0, 512)])

    def s1_body(v, carry):
        q = seg * 512 + v * L + iota
        pidx = ((q >> 6) << 8) + ((q & 63) << 1)
        p = plsc.load_gather(ps_v, [pidx]) + 32
        d = d_v[pl.ds(v * L, L)]
        plsc.addupdate_scatter(s_v, [p], d)
        plsc.addupdate_scatter(n_v, [p], ones)
        return carry

    lax.fori_loop(0, 512 // L, s1_body, 0)

    pltpu.sync_copy(d2_hbm.at[b, pl.ds(seg * 128, 128)], d_v.at[pl.ds(0, 128)])

    def s2_body(v, carry):
        q = seg * 128 + v * L + iota
        pidx = ((q >> 5) << 9) + ((q & 31) << 2)
        p = plsc.load_gather(ps_v, [pidx]) + 64
        d = d_v[pl.ds(v * L, L)]
        plsc.addupdate_scatter(s_v, [p], d)
        plsc.addupdate_scatter(n_v, [p], ones)
        return carry

    lax.fori_loop(0, 128 // L, s2_body, 0)

    @pl.when(seg < 4)
    def _():
        pltpu.sync_copy(d3_hbm.at[b, pl.ds(seg * L, L)], d_v.at[pl.ds(0, L)])
        q = seg * L + iota
        pidx = ((q >> 4) << 10) + ((q & 15) << 3)
        p = plsc.load_gather(ps_v, [pidx]) + 96
        d = d_v[pl.ds(0, L)]
        plsc.addupdate_scatter(s_v, [p], d)
        plsc.addupdate_scatter(n_v, [p], ones)

    pltpu.sync_copy(s_v, s_hbm.at[pl.ds(wid * 128, 128)])
    pltpu.sync_copy(n_v, n_hbm.at[pl.ds(wid * 128, 128)])


@functools.partial(
    pl.kernel,
    out_type=jax.ShapeDtypeStruct((L,), jnp.float32),
    mesh=_mesh(),
    scratch_types=[
        pltpu.VMEM((NW * 128,), jnp.float32),
        pltpu.VMEM((NW * 128,), jnp.float32),
        pltpu.VMEM((L,), jnp.float32),
    ],
)
def _finalize_kernel(s_hbm, n_hbm, out_hbm, s_v, n_v, o_v):
    cid = lax.axis_index("c")
    sid = lax.axis_index("s")

    @pl.when(jnp.logical_and(cid == 0, sid == 0))
    def _():
        pltpu.sync_copy(s_hbm, s_v)
        pltpu.sync_copy(n_hbm, n_v)
        iota = lax.iota(jnp.int32, (L,))
        chans = [128.0, 256.0, 512.0, 512.0]
        wts = [1.0, 2.0, 3.0, 4.0]
        total = jnp.zeros((L,), jnp.float32)
        for s in range(4):
            for half in range(2):
                off = s * 32 + half * L
                ssum = jnp.zeros((L,), jnp.float32)
                nsum = jnp.zeros((L,), jnp.float32)
                for w in range(NW):
                    ssum = ssum + s_v[pl.ds(w * 128 + off, L)]
                    nsum = nsum + n_v[pl.ds(w * 128 + off, L)]
                cl = half * L + iota
                coef = jnp.where(
                    cl == 0, jnp.float32(NUM_OLD / NUM_CLASS),
                    jnp.where(cl <= NUM_OLD, jnp.float32(1.0),
                              jnp.float32(0.0)))
                denom = jnp.maximum(nsum * chans[s], 1.0)
                term = jnp.where(nsum > 0, coef * ssum / denom,
                                 jnp.float32(0.0))
                total = total + wts[s] * term
        o_v[...] = jnp.full((L,), jnp.sum(total) * 0.25)
        pltpu.sync_copy(o_v, out_hbm)


def kernel(labels, outputs_old, f0_old, f1_old, f2_old, f3_old,
           f0, f1, f2, f3, num_class, num_old_class):
    del num_class, num_old_class
    labels = labels.astype(jnp.int32)
    oo2 = outputs_old.reshape(B * OC * HF, HF)
    lab2 = labels.reshape(B * HF, HF)
    pseudo = _pseudo_kernel(oo2, lab2)

    d0 = _dsum(f0, f0_old, 16)
    d1 = _dsum(f1, f1_old, 64)
    d2 = _dsum(f2, f2_old, 128)
    d3 = _dsum(f3, f3_old, 512)

    s_part, n_part = _bin_kernel(pseudo.reshape(B, H0 * H0), d0, d1, d2, d3)
    out16 = _finalize_kernel(s_part, n_part)
    return out16[0]


# speedup vs baseline: 1.8995x; 1.8995x over previous
import functools

import jax
import jax.numpy as jnp
from jax import lax
from jax.experimental import pallas as pl
from jax.experimental.pallas import tpu as pltpu
from jax.experimental.pallas import tpu_sc as plsc

NC, NS, L = 2, 16, 16
NW = NC * NS

B = 4
HF = 512
OC = 16
H0 = 128
NUM_CLASS = 21
NUM_OLD = 16

_mesh = lambda: plsc.VectorSubcoreMesh(core_axis_name="c", subcore_axis_name="s")


def _dsum_body(f_ref, g_ref, o_ref):
    c = pl.program_id(1)

    @pl.when(c == 0)
    def _():
        o_ref[...] = jnp.zeros_like(o_ref)

    x = f_ref[...] - g_ref[...]
    o_ref[...] += jnp.sum(x * x, axis=1, keepdims=True)


def _dsum(f, f_old, c_blk):
    b, c, h, w = f.shape
    hw = h * w
    f2 = f.reshape(b, c, hw)
    g2 = f_old.reshape(b, c, hw)
    out = pl.pallas_call(
        _dsum_body,
        grid=(b, c // c_blk),
        in_specs=[
            pl.BlockSpec((1, c_blk, hw), lambda i, j: (i, j, 0)),
            pl.BlockSpec((1, c_blk, hw), lambda i, j: (i, j, 0)),
        ],
        out_specs=pl.BlockSpec((1, 1, hw), lambda i, j: (i, 0, 0)),
        out_shape=jax.ShapeDtypeStruct((b, 1, hw), jnp.float32),
        compiler_params=pltpu.CompilerParams(
            dimension_semantics=("parallel", "arbitrary")),
    )(f2, g2)
    return out.reshape(b, hw)


@functools.partial(
    pl.kernel,
    out_type=jax.ShapeDtypeStruct((B * H0, H0), jnp.int32),
    mesh=_mesh(),
    scratch_types=[
        pltpu.VMEM((OC, HF), jnp.float32),
        pltpu.VMEM((1, HF), jnp.int32),
        pltpu.VMEM((1, H0), jnp.int32),
        pltpu.SemaphoreType.DMA,
        pltpu.SemaphoreType.DMA,
    ],
    compiler_params=pltpu.CompilerParams(use_tc_tiling_on_sc=False, needs_layout_passes=False),
)
def _pseudo_kernel(oo_hbm, lab_hbm, out_hbm, rows_v, lab_v, ps_v, sem1, sem2):
    wid = lax.axis_index("s") * NC + lax.axis_index("c")
    iota = lax.iota(jnp.int32, L)
    ntask = (B * H0) // NW

    def task(k, carry):
        t = wid * ntask + k
        b = t // H0
        i = t % H0
        ridx = b * (OC * HF) + 4 * i + HF * iota
        cp1 = pltpu.async_copy(oo_hbm.at[ridx], rows_v, sem1)
        cp2 = pltpu.async_copy(lab_hbm.at[pl.ds(b * HF + 4 * i, 1)], lab_v, sem2)
        cp1.wait()
        cp2.wait()
        for j0 in range(0, H0, L):
            cols = 4 * (j0 + iota)
            best = None
            bidx = jnp.zeros((L,), jnp.int32)
            for c in range(OC):
                rv = plsc.load_gather(
                    rows_v, [jnp.full((L,), c, jnp.int32), cols])
                v = jnp.where(rv < 0.5, jnp.float32(0.0), rv)
                if c == 0:
                    best = v
                else:
                    take = v > best
                    best = jnp.where(take, v, best)
                    bidx = jnp.where(take, c, bidx)
            lv = plsc.load_gather(lab_v, [jnp.zeros((L,), jnp.int32), cols])
            ps_v[0, pl.ds(j0, L)] = jnp.where(lv == 0, bidx, lv)
        pltpu.sync_copy(ps_v, out_hbm.at[pl.ds(t, 1)])
        return carry

    lax.fori_loop(0, ntask, task, 0)


@functools.partial(
    pl.kernel,
    out_type=(
        jax.ShapeDtypeStruct((NW * 128,), jnp.float32),
        jax.ShapeDtypeStruct((NW * 128,), jnp.float32),
    ),
    mesh=_mesh(),
    scratch_types=[
        pltpu.VMEM((H0 * H0,), jnp.int32),
        pltpu.VMEM((2048,), jnp.float32),
        pltpu.VMEM((128,), jnp.float32),
        pltpu.VMEM((128,), jnp.float32),
    ],
    compiler_params=pltpu.CompilerParams(use_tc_tiling_on_sc=False, needs_layout_passes=False),
)
def _bin_kernel(ps_hbm, d0_hbm, d1_hbm, d2_hbm, d3_hbm,
                s_hbm, n_hbm, ps_v, d_v, s_v, n_v):
    wid = lax.axis_index("s") * NC + lax.axis_index("c")
    b = wid // 8
    seg = wid % 8
    zero = jnp.zeros((L,), jnp.float32)
    ones = jnp.ones((L,), jnp.float32)
    iota = lax.iota(jnp.int32, L)
    for j in range(8):
        s_v[pl.ds(j * L, L)] = zero
        n_v[pl.ds(j * L, L)] = zero
    pltpu.sync_copy(ps_hbm.at[b], ps_v)

    pltpu.sync_copy(d0_hbm.at[b, pl.ds(seg * 2048, 2048)], d_v)

    def s0_body(v, carry):
        p = ps_v[pl.ds(seg * 2048 + v * L, L)]
        d = d_v[pl.ds(v * L, L)]
        plsc.addupdate_scatter(s_v, [p], d)
        plsc.addupdate_scatter(n_v, [p], ones)
        return carry

    lax.fori_loop(0, 2048 // L, s0_body, 0)

    pltpu.sync_copy(d1_hbm.at[b, pl.ds(seg * 512, 512)], d_v.at[pl.ds(0, 512)])

    def s1_body(v, carry):
        q = seg * 512 + v * L + iota
        pidx = ((q >> 6) << 8) + ((q & 63) << 1)
        p = plsc.load_gather(ps_v, [pidx]) + 32
        d = d_v[pl.ds(v * L, L)]
        plsc.addupdate_scatter(s_v, [p], d)
        plsc.addupdate_scatter(n_v, [p], ones)
        return carry

    lax.fori_loop(0, 512 // L, s1_body, 0)

    pltpu.sync_copy(d2_hbm.at[b, pl.ds(seg * 128, 128)], d_v.at[pl.ds(0, 128)])

    def s2_body(v, carry):
        q = seg * 128 + v * L + iota
        pidx = ((q >> 5) << 9) + ((q & 31) << 2)
        p = plsc.load_gather(ps_v, [pidx]) + 64
        d = d_v[pl.ds(v * L, L)]
        plsc.addupdate_scatter(s_v, [p], d)
        plsc.addupdate_scatter(n_v, [p], ones)
        return carry

    lax.fori_loop(0, 128 // L, s2_body, 0)

    @pl.when(seg < 4)
    def _():
        pltpu.sync_copy(d3_hbm.at[b, pl.ds(seg * L, L)], d_v.at[pl.ds(0, L)])
        q = seg * L + iota
        pidx = ((q >> 4) << 10) + ((q & 15) << 3)
        p = plsc.load_gather(ps_v, [pidx]) + 96
        d = d_v[pl.ds(0, L)]
        plsc.addupdate_scatter(s_v, [p], d)
        plsc.addupdate_scatter(n_v, [p], ones)

    pltpu.sync_copy(s_v, s_hbm.at[pl.ds(wid * 128, 128)])
    pltpu.sync_copy(n_v, n_hbm.at[pl.ds(wid * 128, 128)])


@functools.partial(
    pl.kernel,
    out_type=jax.ShapeDtypeStruct((L,), jnp.float32),
    mesh=_mesh(),
    scratch_types=[
        pltpu.VMEM((NW * 128,), jnp.float32),
        pltpu.VMEM((NW * 128,), jnp.float32),
        pltpu.VMEM((L,), jnp.float32),
    ],
    compiler_params=pltpu.CompilerParams(use_tc_tiling_on_sc=False, needs_layout_passes=False),
)
def _finalize_kernel(s_hbm, n_hbm, out_hbm, s_v, n_v, o_v):
    cid = lax.axis_index("c")
    sid = lax.axis_index("s")

    @pl.when(jnp.logical_and(cid == 0, sid == 0))
    def _():
        pltpu.sync_copy(s_hbm, s_v)
        pltpu.sync_copy(n_hbm, n_v)
        iota = lax.iota(jnp.int32, L)
        chans = [128.0, 256.0, 512.0, 512.0]
        wts = [1.0, 2.0, 3.0, 4.0]
        total = jnp.zeros((L,), jnp.float32)
        for s in range(4):
            for half in range(2):
                off = s * 32 + half * L
                ssum = jnp.zeros((L,), jnp.float32)
                nsum = jnp.zeros((L,), jnp.float32)
                for w in range(NW):
                    ssum = ssum + s_v[pl.ds(w * 128 + off, L)]
                    nsum = nsum + n_v[pl.ds(w * 128 + off, L)]
                cl = half * L + iota
                coef = jnp.where(
                    cl == 0, jnp.float32(NUM_OLD / NUM_CLASS),
                    jnp.where(cl <= NUM_OLD, jnp.float32(1.0),
                              jnp.float32(0.0)))
                denom = jnp.maximum(nsum * chans[s], 1.0)
                term = jnp.where(nsum > 0, coef * ssum / denom,
                                 jnp.float32(0.0))
                total = total + wts[s] * term
        o_v[...] = jnp.full((L,), jnp.sum(total) * 0.25)
        pltpu.sync_copy(o_v, out_hbm)


def kernel(labels, outputs_old, f0_old, f1_old, f2_old, f3_old,
           f0, f1, f2, f3, num_class, num_old_class):
    del num_class, num_old_class
    labels = labels.astype(jnp.int32)
    oo2 = outputs_old.reshape(B * OC * HF, HF)
    lab2 = labels.reshape(B * HF, HF)
    pseudo = _pseudo_kernel(oo2, lab2)

    d0 = _dsum(f0, f0_old, 16)
    d1 = _dsum(f1, f1_old, 64)
    d2 = _dsum(f2, f2_old, 128)
    d3 = _dsum(f3, f3_old, 512)

    s_part, n_part = _bin_kernel(pseudo.reshape(B, H0 * H0), d0, d1, d2, d3)
    out16 = _finalize_kernel(s_part, n_part)
    return out16[0]
